# baseline (speedup 1.0000x reference)
import functools
import numpy as np
import jax
import jax.numpy as jnp
from jax import lax
from jax.experimental import pallas as pl
from jax.experimental.pallas import tpu as pltpu
from jax.experimental.pallas import tpu_sc as plsc

EMB = 64
RF = 20
CUTOFF = 5.0


def _k0_body(z_ref, zt_ref, w0_ref, w1_ref, p01_ref):
    bn = z_ref.shape[0]
    oh = (z_ref[...] == lax.broadcasted_iota(jnp.int32, (bn, 128), 1)
          ).astype(jnp.float32)
    t0 = jnp.dot(zt_ref[...], w0_ref[...], preferred_element_type=jnp.float32)
    t1 = jnp.dot(zt_ref[...], w1_ref[...], preferred_element_type=jnp.float32)
    p01_ref[:, 0:EMB] = jnp.dot(oh[:, :100], t0,
                                preferred_element_type=jnp.float32)
    p01_ref[:, EMB:2 * EMB] = jnp.dot(oh[:, :100], t1,
                                      preferred_element_type=jnp.float32)


def _species_tables(Z2, z_table, Wz0T, Wz1T):
    N = Z2.shape[0]
    BN = 2000
    return pl.pallas_call(
        _k0_body,
        grid=(N // BN,),
        in_specs=[
            pl.BlockSpec((BN, 1), lambda i: (i, 0)),
            pl.BlockSpec((100, EMB), lambda i: (0, 0)),
            pl.BlockSpec((EMB, EMB), lambda i: (0, 0)),
            pl.BlockSpec((EMB, EMB), lambda i: (0, 0)),
        ],
        out_specs=pl.BlockSpec((BN, 2 * EMB), lambda i: (i, 0)),
        out_shape=jax.ShapeDtypeStruct((N, 2 * EMB), jnp.float32),
    )(Z2, z_table, Wz0T, Wz1T)


def _make_k1(E):
    NG = E // 128
    PER = -(-NG // 32)
    mesh = plsc.VectorSubcoreMesh(core_axis_name="c", subcore_axis_name="s")

    @functools.partial(
        pl.kernel,
        mesh=mesh,
        compiler_params=pltpu.CompilerParams(needs_layout_passes=False),
        out_type=jax.ShapeDtypeStruct((E, 2 * EMB), jnp.float32),
        scratch_types=[
            pltpu.VMEM((128,), jnp.int32),
            pltpu.VMEM((128,), jnp.int32),
            pltpu.VMEM((128, 2 * EMB), jnp.float32),
            pltpu.VMEM((128, 2 * EMB), jnp.float32),
            pltpu.SemaphoreType.DMA,
            pltpu.SemaphoreType.DMA,
        ],
    )
    def k1(p01_hbm, ia_hbm, ib_hbm, out_hbm,
           ia_v, ib_v, r0_v, r1_v, sem0, sem1):
        cid = lax.axis_index("c")
        sid = lax.axis_index("s")
        wid = sid * 2 + cid

        def grp(g, carry):
            gg = jnp.minimum(wid * PER + g, NG - 1)
            base = gg * 128
            pltpu.sync_copy(ia_hbm.at[pl.ds(base, 128)], ia_v)
            pltpu.sync_copy(ib_hbm.at[pl.ds(base, 128)], ib_v)
            cp0 = pltpu.async_copy(p01_hbm.at[ia_v], r0_v, sem0)
            cp1 = pltpu.async_copy(p01_hbm.at[ib_v], r1_v, sem1)
            cp0.wait()
            cp1.wait()

            def addrow(j, c2):
                for m in range(4):
                    sl = pl.ds(m * 16, 16)
                    slb = pl.ds(EMB + m * 16, 16)
                    r0_v[j, sl] = r0_v[j, sl] + r1_v[j, slb]
                return c2

            lax.fori_loop(0, 128, addrow, 0)
            pltpu.sync_copy(r0_v, out_hbm.at[pl.ds(base, 128), :])
            return carry

        lax.fori_loop(0, PER, grp, 0)

    return k1


def _k2_body(beta, nvt_ref, hs_ref, wr_ref, br_ref, mu_ref, i0_ref, out_ref):
    x = nvt_ref[0:1, :]
    y = nvt_ref[1:2, :]
    z = nvt_ref[2:3, :]
    r2 = x * x + y * y + z * z
    inv = lax.rsqrt(r2)
    r = r2 * inv
    vx = x * inv
    vy = y * inv
    vz = z * inv
    er = jnp.exp(-r)
    hR = jnp.exp(-beta * (er - mu_ref[...]) ** 2)
    env = 0.5 * (jnp.cos(jnp.pi * r / CUTOFF) + 1.0)
    env = jnp.where(r < CUTOFF, env, 0.0)
    q = jnp.dot(wr_ref[...], hR, preferred_element_type=jnp.float32)
    q = (q + br_ref[...]) * env
    qrows = lax.transpose(q, (1, 0))
    hs = hs_ref[:, 0:EMB]
    h3 = jnp.concatenate([hs] * 3, axis=1)
    geom = jnp.concatenate(
        [vx, vy, vz, vx * vx, vy * vy, vz * vz,
         vx * vy, vx * vz, vy * vz], axis=0)
    grows = lax.transpose(geom, (1, 0))
    out_ref[:, 0:192] = qrows * h3
    out_ref[:, 192:201] = grows
    out_ref[:, 201:202] = lax.bitcast_convert_type(i0_ref[...], jnp.float32)
    out_ref[:, 202:256] = jnp.zeros_like(out_ref[:, 202:256])


def _edge_payload(nvT, Hsum, Wr, br2, mu2, i02):
    E = Hsum.shape[0]
    BE = 1280
    body = functools.partial(
        _k2_body, float((2.0 / RF * (1.0 - np.exp(-CUTOFF))) ** -2))
    return pl.pallas_call(
        body,
        grid=(E // BE,),
        in_specs=[
            pl.BlockSpec((4, BE), lambda i: (0, i)),
            pl.BlockSpec((BE, 2 * EMB), lambda i: (i, 0)),
            pl.BlockSpec((3 * EMB, RF), lambda i: (0, 0)),
            pl.BlockSpec((3 * EMB, 1), lambda i: (0, 0)),
            pl.BlockSpec((RF, 1), lambda i: (0, 0)),
            pl.BlockSpec((BE, 1), lambda i: (i, 0)),
        ],
        out_specs=pl.BlockSpec((BE, 256), lambda i: (i, 0)),
        out_shape=jax.ShapeDtypeStruct((E, 256), jnp.float32),
    )(nvT, Hsum, Wr, br2, mu2, i02)


NCHUNK = 128
NCHKS = 79
NCHP = 96
CAPB = 160000


def _pos_body(i0_ref, pos_ref, cnt_ref, carry):
    be = i0_ref.shape[0]

    @pl.when(pl.program_id(0) == 0)
    def _():
        carry[...] = jnp.zeros_like(carry)

    bucket = lax.shift_right_logical(i0_ref[...], 7)
    cols = lax.broadcasted_iota(jnp.int32, (be, NCHP), 1)
    oh = (bucket == cols).astype(jnp.float32)
    rows_i = lax.broadcasted_iota(jnp.int32, (be, be), 0)
    cols_i = lax.broadcasted_iota(jnp.int32, (be, be), 1)
    tri = (rows_i > cols_i).astype(jnp.float32)
    rank = jnp.sum(jnp.dot(tri, oh, preferred_element_type=jnp.float32) * oh,
                   axis=1, keepdims=True)
    cpe = jnp.sum(carry[0:1, :] * oh, axis=1, keepdims=True)
    pos_ref[...] = bucket * CAPB + (rank + cpe).astype(jnp.int32)
    carry[0:1, :] = carry[0:1, :] + jnp.sum(oh, axis=0, keepdims=True)
    cnt_ref[...] = jnp.broadcast_to(carry[0:1, :], cnt_ref.shape)


def _bin_positions(i02):
    E = i02.shape[0]
    BE = 800
    return pl.pallas_call(
        _pos_body,
        grid=(E // BE,),
        in_specs=[pl.BlockSpec((BE, 1), lambda i: (i, 0))],
        out_specs=[
            pl.BlockSpec((BE, 1), lambda i: (i, 0)),
            pl.BlockSpec((8, NCHP), lambda i: (0, 0)),
        ],
        out_shape=[
            jax.ShapeDtypeStruct((E, 1), jnp.int32),
            jax.ShapeDtypeStruct((8, NCHP), jnp.float32),
        ],
        scratch_shapes=[pltpu.VMEM((8, NCHP), jnp.float32)],
    )(i02)


def _make_bin_scatter(E):
    NG = E // 64
    PER = -(-NG // 32)
    mesh = plsc.VectorSubcoreMesh(core_axis_name="c", subcore_axis_name="s")

    @functools.partial(
        pl.kernel,
        mesh=mesh,
        compiler_params=pltpu.CompilerParams(needs_layout_passes=False),
        out_type=jax.ShapeDtypeStruct((NCHKS * CAPB,), jnp.int32),
        scratch_types=[
            pltpu.VMEM((64,), jnp.int32),
            pltpu.VMEM((64,), jnp.int32),
            pltpu.SemaphoreType.DMA,
        ],
    )
    def kbin(pos_hbm, bins_hbm, pos_v, eid_v, sem):
        cid = lax.axis_index("c")
        sid = lax.axis_index("s")
        wid = sid * 2 + cid
        lane = lax.broadcasted_iota(jnp.int32, (16,), 0)

        def grp(g, carry):
            gg = jnp.minimum(wid * PER + g, NG - 1)
            base = gg * 64
            pltpu.sync_copy(pos_hbm.at[pl.ds(base, 64)], pos_v)
            for k in range(4):
                eid_v[pl.ds(k * 16, 16)] = lane + (base + k * 16)
            pltpu.async_copy(eid_v, bins_hbm.at[pos_v], sem).wait()
            return carry

        lax.fori_loop(0, PER, grp, 0)

    return kbin


def _make_k3(E, N):
    MAXG = E // 128
    mesh = plsc.VectorSubcoreMesh(core_axis_name="c", subcore_axis_name="s")

    @functools.partial(
        pl.kernel,
        mesh=mesh,
        compiler_params=pltpu.CompilerParams(needs_layout_passes=False),
        out_type=jax.ShapeDtypeStruct((NCHKS * NCHUNK, 640), jnp.float32),
        scratch_types=[
            pltpu.VMEM((128, 256), jnp.float32),
            pltpu.VMEM((128,), jnp.int32),
            pltpu.VMEM((128,), jnp.int32),
            pltpu.VMEM((NCHP,), jnp.int32),
            pltpu.VMEM((NCHUNK + 1, 640), jnp.float32),
            pltpu.SemaphoreType.DMA,
        ],
    )
    def k3(pay_hbm, bins_hbm, cnt_hbm, zeros_hbm, acc_hbm,
           pay_v, ids_v, idx_v, cnt_v, acc_v, sem):
        cid = lax.axis_index("c")
        sid = lax.axis_index("s")
        wid = sid * 2 + cid
        pltpu.sync_copy(cnt_hbm, cnt_v)
        lane = lax.broadcasted_iota(jnp.int32, (16,), 0)

        for ph in range(-(-NCHKS // 32)):
            chunk = ph * 32 + wid

            @pl.when(chunk < NCHKS)
            def _():
                abase = chunk * NCHUNK
                cnt = plsc.load_gather(cnt_v, [jnp.full((16,), chunk)])[0]
                pltpu.sync_copy(zeros_hbm, acc_v.at[pl.ds(0, NCHUNK)])

                def grp(g, carry):
                    gbase = g * 128

                    @pl.when(gbase < cnt)
                    def _():
                        pltpu.sync_copy(
                            bins_hbm.at[pl.ds(chunk * CAPB + gbase, 128)],
                            ids_v)
                        for k in range(8):
                            sl = pl.ds(k * 16, 16)
                            gpos = lane + (gbase + k * 16)
                            okv = gpos < jnp.full((16,), cnt)
                            ids_v[sl] = jnp.where(okv, ids_v[sl], 0)
                        cp = pltpu.async_copy(pay_hbm.at[ids_v], pay_v, sem)
                        cp.wait()
                        for k in range(8):
                            sl = pl.ds(k * 16, 16)
                            gpos = lane + (gbase + k * 16)
                            okv = gpos < jnp.full((16,), cnt)
                            i0g = plsc.bitcast(
                                plsc.load_gather(
                                    pay_v,
                                    [lane + k * 16, jnp.full((16,), 201)]),
                                jnp.int32)
                            idx_v[sl] = jnp.where(okv, i0g - abase, NCHUNK)

                        def edge(j, c2):
                            li = plsc.load_gather(idx_v, [jnp.full((16,), j)])
                            gvec = pay_v[j, pl.ds(192, 16)]
                            gv = [jnp.full((16,), gvec[t]) for t in range(9)]
                            cmid = [pay_v[j, pl.ds(64 + m * 16, 16)]
                                    for m in range(4)]
                            chi = [pay_v[j, pl.ds(128 + m * 16, 16)]
                                   for m in range(4)]
                            for m in range(4):
                                plsc.addupdate_scatter(
                                    acc_v,
                                    [li, lane + m * 16],
                                    pay_v[j, pl.ds(m * 16, 16)])
                            for p in range(1, 4):
                                for m in range(4):
                                    plsc.addupdate_scatter(
                                        acc_v,
                                        [li, lane + (p * 64 + m * 16)],
                                        cmid[m] * gv[p - 1])
                            for p in range(4, 10):
                                for m in range(4):
                                    plsc.addupdate_scatter(
                                        acc_v,
                                        [li, lane + (p * 64 + m * 16)],
                                        chi[m] * gv[p - 1])
                            return c2

                        lax.fori_loop(0, 128, edge, 0)
                    return carry

                lax.fori_loop(0, MAXG, grp, 0)
                pltpu.sync_copy(acc_v.at[pl.ds(0, NCHUNK)],
                                acc_hbm.at[pl.ds(abase, NCHUNK), :])

    return k3


def _k4_body(acc_ref, g_ref, b_ref, w1t_ref, b1_ref, w2t_ref, b2_ref,
             wit_ref, wat_ref, wst_ref, out_ref):
    A = acc_ref[...]
    x = [A[:, 64 * p:64 * (p + 1)] for p in range(10)]
    trace = x[4] + x[5] + x[6]
    dd = x[0] - trace * (1.0 / 3.0)
    norms = ((dd + x[4]) ** 2 + (dd + x[5]) ** 2 + (dd + x[6]) ** 2
             + 2.0 * (x[7] ** 2 + x[3] ** 2)
             + 2.0 * (x[8] ** 2 + x[2] ** 2)
             + 2.0 * (x[9] ** 2 + x[1] ** 2))
    mean = jnp.mean(norms, axis=-1, keepdims=True)
    var = jnp.mean((norms - mean) ** 2, axis=-1, keepdims=True)
    nn = (norms - mean) * lax.rsqrt(var + 1e-5) * g_ref[...] + b_ref[...]
    h1 = jnp.dot(nn, w1t_ref[...], preferred_element_type=jnp.float32)
    h1 = h1 + b1_ref[...]
    h1 = h1 * (1.0 / (1.0 + jnp.exp(-h1)))
    co = jnp.dot(h1, w2t_ref[...], preferred_element_type=jnp.float32)
    co = co + b2_ref[...]
    co = co * (1.0 / (1.0 + jnp.exp(-co)))
    cI2 = co[:, 0:64]
    cA2 = co[:, 64:128]
    cS2 = co[:, 128:192]
    ws = [wit_ref, wat_ref, wat_ref, wat_ref] + [wst_ref] * 6
    t = [jnp.dot(x[p], ws[p][...], preferred_element_type=jnp.float32)
         for p in range(10)]
    tr3 = (t[4] + t[5] + t[6]) * (1.0 / 3.0)
    diag = cI2 * t[0]
    out_ref[0] = diag + cS2 * (t[4] - tr3)
    out_ref[1] = cS2 * t[7] - cA2 * t[3]
    out_ref[2] = cS2 * t[8] + cA2 * t[2]
    out_ref[3] = cS2 * t[7] + cA2 * t[3]
    out_ref[4] = diag + cS2 * (t[5] - tr3)
    out_ref[5] = cS2 * t[9] - cA2 * t[1]
    out_ref[6] = cS2 * t[8] - cA2 * t[2]
    out_ref[7] = cS2 * t[9] + cA2 * t[1]
    out_ref[8] = diag + cS2 * (t[6] - tr3)


def _atom_stage(N, Acc, ln_g2, ln_b2, W1T, b12, W2T, b22, WIT, WAT, WST):
    BN = 1000
    return pl.pallas_call(
        _k4_body,
        grid=(N // BN,),
        in_specs=[
            pl.BlockSpec((BN, 640), lambda i: (i, 0)),
            pl.BlockSpec((1, EMB), lambda i: (0, 0)),
            pl.BlockSpec((1, EMB), lambda i: (0, 0)),
            pl.BlockSpec((EMB, 2 * EMB), lambda i: (0, 0)),
            pl.BlockSpec((1, 2 * EMB), lambda i: (0, 0)),
            pl.BlockSpec((2 * EMB, 3 * EMB), lambda i: (0, 0)),
            pl.BlockSpec((1, 3 * EMB), lambda i: (0, 0)),
            pl.BlockSpec((EMB, EMB), lambda i: (0, 0)),
            pl.BlockSpec((EMB, EMB), lambda i: (0, 0)),
            pl.BlockSpec((EMB, EMB), lambda i: (0, 0)),
        ],
        out_specs=pl.BlockSpec((9, BN, EMB), lambda i: (0, i, 0)),
        out_shape=jax.ShapeDtypeStruct((9, N, EMB), jnp.float32),
    )(Acc, ln_g2, ln_b2, W1T, b12, W2T, b22, WIT, WAT, WST)


def kernel(neighbour_vectors, Z, neighbour_index, z_table, Wz, Wr, br,
           ln_g, ln_b, W1, b1, W2, b2, WI, WA, WS):
    E = neighbour_vectors.shape[0]
    N = Z.shape[0]

    nvT = jnp.zeros((4, E), jnp.float32).at[:3].set(neighbour_vectors.T)
    i0 = neighbour_index[0]
    i1 = neighbour_index[1]
    idxA = jnp.concatenate([i0[0::2], i1[0::2]])
    idxB = jnp.concatenate([i0[1::2], i1[1::2]])

    P01 = _species_tables(Z[:, None], z_table,
                          Wz[:, :EMB].T, Wz[:, EMB:].T)
    Hsum = _make_k1(E)(P01, idxA, idxB)
    mu2 = jnp.linspace(jnp.exp(-CUTOFF), 1.0, RF,
                       dtype=jnp.float32)[:, None]
    i02 = i0[:, None]
    pay = _edge_payload(nvT, Hsum, Wr, br[:, None], mu2, i02)
    pos, cntf = _bin_positions(i02)
    bins = _make_bin_scatter(E)(pos[:, 0])
    zeros = jnp.zeros((NCHUNK, 640), jnp.float32)
    Acc = _make_k3(E, N)(pay, bins, cntf[0].astype(jnp.int32), zeros)
    X9 = _atom_stage(N, Acc, ln_g[None, :], ln_b[None, :], W1.T, b1[None, :],
                     W2.T, b2[None, :], WI.T, WA.T, WS.T)
    return jnp.transpose(X9, (1, 2, 0)).reshape(N, EMB, 3, 3)


# speedup vs baseline: 1.0627x; 1.0627x over previous
import functools
import numpy as np
import jax
import jax.numpy as jnp
from jax import lax
from jax.experimental import pallas as pl
from jax.experimental.pallas import tpu as pltpu
from jax.experimental.pallas import tpu_sc as plsc

EMB = 64
RF = 20
CUTOFF = 5.0


def _k0_body(z_ref, zt_ref, w0_ref, w1_ref, p01_ref):
    bn = z_ref.shape[0]
    oh = (z_ref[...] == lax.broadcasted_iota(jnp.int32, (bn, 128), 1)
          ).astype(jnp.float32)
    t0 = jnp.dot(zt_ref[...], w0_ref[...], preferred_element_type=jnp.float32)
    t1 = jnp.dot(zt_ref[...], w1_ref[...], preferred_element_type=jnp.float32)
    p01_ref[:, 0:EMB] = jnp.dot(oh[:, :100], t0,
                                preferred_element_type=jnp.float32)
    p01_ref[:, EMB:2 * EMB] = jnp.dot(oh[:, :100], t1,
                                      preferred_element_type=jnp.float32)


def _species_tables(Z2, z_table, Wz0T, Wz1T):
    N = Z2.shape[0]
    BN = 2000
    return pl.pallas_call(
        _k0_body,
        grid=(N // BN,),
        in_specs=[
            pl.BlockSpec((BN, 1), lambda i: (i, 0)),
            pl.BlockSpec((100, EMB), lambda i: (0, 0)),
            pl.BlockSpec((EMB, EMB), lambda i: (0, 0)),
            pl.BlockSpec((EMB, EMB), lambda i: (0, 0)),
        ],
        out_specs=pl.BlockSpec((BN, 2 * EMB), lambda i: (i, 0)),
        out_shape=jax.ShapeDtypeStruct((N, 2 * EMB), jnp.float32),
    )(Z2, z_table, Wz0T, Wz1T)


def _make_k1(E):
    NG = E // 128
    PER = -(-NG // 32)
    mesh = plsc.VectorSubcoreMesh(core_axis_name="c", subcore_axis_name="s")

    @functools.partial(
        pl.kernel,
        mesh=mesh,
        compiler_params=pltpu.CompilerParams(needs_layout_passes=False),
        out_type=jax.ShapeDtypeStruct((E, 2 * EMB), jnp.float32),
        scratch_types=[
            pltpu.VMEM((128,), jnp.int32),
            pltpu.VMEM((128,), jnp.int32),
            pltpu.VMEM((128, 2 * EMB), jnp.float32),
            pltpu.VMEM((128, 2 * EMB), jnp.float32),
            pltpu.SemaphoreType.DMA,
            pltpu.SemaphoreType.DMA,
        ],
    )
    def k1(p01_hbm, ia_hbm, ib_hbm, out_hbm,
           ia_v, ib_v, r0_v, r1_v, sem0, sem1):
        cid = lax.axis_index("c")
        sid = lax.axis_index("s")
        wid = sid * 2 + cid

        def grp(g, carry):
            gg = jnp.minimum(wid * PER + g, NG - 1)
            base = gg * 128
            pltpu.sync_copy(ia_hbm.at[pl.ds(base, 128)], ia_v)
            pltpu.sync_copy(ib_hbm.at[pl.ds(base, 128)], ib_v)
            cp0 = pltpu.async_copy(p01_hbm.at[ia_v], r0_v, sem0)
            cp1 = pltpu.async_copy(p01_hbm.at[ib_v], r1_v, sem1)
            cp0.wait()
            cp1.wait()

            def addrow(j, c2):
                for m in range(4):
                    sl = pl.ds(m * 16, 16)
                    slb = pl.ds(EMB + m * 16, 16)
                    r0_v[j, sl] = r0_v[j, sl] + r1_v[j, slb]
                return c2

            lax.fori_loop(0, 128, addrow, 0)
            pltpu.sync_copy(r0_v, out_hbm.at[pl.ds(base, 128), :])
            return carry

        lax.fori_loop(0, PER, grp, 0)

    return k1


def _k2_body(beta, nvt_ref, hs_ref, wr_ref, br_ref, mu_ref, i0_ref, out_ref):
    x = nvt_ref[0:1, :]
    y = nvt_ref[1:2, :]
    z = nvt_ref[2:3, :]
    r2 = x * x + y * y + z * z
    inv = lax.rsqrt(r2)
    r = r2 * inv
    vx = x * inv
    vy = y * inv
    vz = z * inv
    er = jnp.exp(-r)
    hR = jnp.exp(-beta * (er - mu_ref[...]) ** 2)
    env = 0.5 * (jnp.cos(jnp.pi * r / CUTOFF) + 1.0)
    env = jnp.where(r < CUTOFF, env, 0.0)
    q = jnp.dot(wr_ref[...], hR, preferred_element_type=jnp.float32)
    q = (q + br_ref[...]) * env
    qrows = lax.transpose(q, (1, 0))
    hs = hs_ref[:, 0:EMB]
    h3 = jnp.concatenate([hs] * 3, axis=1)
    geom = jnp.concatenate(
        [vx, vy, vz, vx * vx, vy * vy, vz * vz,
         vx * vy, vx * vz, vy * vz], axis=0)
    grows = lax.transpose(geom, (1, 0))
    out_ref[:, 0:192] = qrows * h3
    out_ref[:, 192:201] = grows
    out_ref[:, 201:202] = lax.bitcast_convert_type(i0_ref[...], jnp.float32)
    out_ref[:, 202:256] = jnp.zeros_like(out_ref[:, 202:256])


def _edge_payload(nvT, Hsum, Wr, br2, mu2, i02):
    E = Hsum.shape[0]
    BE = 1280
    body = functools.partial(
        _k2_body, float((2.0 / RF * (1.0 - np.exp(-CUTOFF))) ** -2))
    return pl.pallas_call(
        body,
        grid=(E // BE,),
        in_specs=[
            pl.BlockSpec((4, BE), lambda i: (0, i)),
            pl.BlockSpec((BE, 2 * EMB), lambda i: (i, 0)),
            pl.BlockSpec((3 * EMB, RF), lambda i: (0, 0)),
            pl.BlockSpec((3 * EMB, 1), lambda i: (0, 0)),
            pl.BlockSpec((RF, 1), lambda i: (0, 0)),
            pl.BlockSpec((BE, 1), lambda i: (i, 0)),
        ],
        out_specs=pl.BlockSpec((BE, 256), lambda i: (i, 0)),
        out_shape=jax.ShapeDtypeStruct((E, 256), jnp.float32),
    )(nvT, Hsum, Wr, br2, mu2, i02)


NCHUNK = 128
NCHKS = 79
NCHP = 96
CAPB = 160000


def _pos_body(i0_ref, pos_ref, cnt_ref, carry):
    be = i0_ref.shape[0]

    @pl.when(pl.program_id(0) == 0)
    def _():
        carry[...] = jnp.zeros_like(carry)

    bucket = lax.shift_right_logical(i0_ref[...], 7)
    cols = lax.broadcasted_iota(jnp.int32, (be, NCHP), 1)
    oh = (bucket == cols).astype(jnp.float32)
    rows_i = lax.broadcasted_iota(jnp.int32, (be, be), 0)
    cols_i = lax.broadcasted_iota(jnp.int32, (be, be), 1)
    tri = (rows_i > cols_i).astype(jnp.float32)
    rank = jnp.sum(jnp.dot(tri, oh, preferred_element_type=jnp.float32) * oh,
                   axis=1, keepdims=True)
    cpe = jnp.sum(carry[0:1, :] * oh, axis=1, keepdims=True)
    pos_ref[...] = bucket * CAPB + (rank + cpe).astype(jnp.int32)
    carry[0:1, :] = carry[0:1, :] + jnp.sum(oh, axis=0, keepdims=True)
    cnt_ref[...] = jnp.broadcast_to(carry[0:1, :], cnt_ref.shape)


def _bin_positions(i02):
    E = i02.shape[0]
    BE = 800
    return pl.pallas_call(
        _pos_body,
        grid=(E // BE,),
        in_specs=[pl.BlockSpec((BE, 1), lambda i: (i, 0))],
        out_specs=[
            pl.BlockSpec((BE, 1), lambda i: (i, 0)),
            pl.BlockSpec((8, NCHP), lambda i: (0, 0)),
        ],
        out_shape=[
            jax.ShapeDtypeStruct((E, 1), jnp.int32),
            jax.ShapeDtypeStruct((8, NCHP), jnp.float32),
        ],
        scratch_shapes=[pltpu.VMEM((8, NCHP), jnp.float32)],
    )(i02)


def _make_bin_scatter(E):
    NG = E // 64
    PER = -(-NG // 32)
    mesh = plsc.VectorSubcoreMesh(core_axis_name="c", subcore_axis_name="s")

    @functools.partial(
        pl.kernel,
        mesh=mesh,
        compiler_params=pltpu.CompilerParams(needs_layout_passes=False),
        out_type=jax.ShapeDtypeStruct((NCHKS * CAPB,), jnp.int32),
        scratch_types=[
            pltpu.VMEM((64,), jnp.int32),
            pltpu.VMEM((64,), jnp.int32),
            pltpu.SemaphoreType.DMA,
        ],
    )
    def kbin(pos_hbm, bins_hbm, pos_v, eid_v, sem):
        cid = lax.axis_index("c")
        sid = lax.axis_index("s")
        wid = sid * 2 + cid
        lane = lax.broadcasted_iota(jnp.int32, (16,), 0)

        def grp(g, carry):
            gg = jnp.minimum(wid * PER + g, NG - 1)
            base = gg * 64
            pltpu.sync_copy(pos_hbm.at[pl.ds(base, 64)], pos_v)
            for k in range(4):
                eid_v[pl.ds(k * 16, 16)] = lane + (base + k * 16)
            pltpu.async_copy(eid_v, bins_hbm.at[pos_v], sem).wait()
            return carry

        lax.fori_loop(0, PER, grp, 0)

    return kbin


def _make_k3(E, N):
    MAXG = E // 64
    mesh = plsc.VectorSubcoreMesh(core_axis_name="c", subcore_axis_name="s")

    @functools.partial(
        pl.kernel,
        mesh=mesh,
        compiler_params=pltpu.CompilerParams(needs_layout_passes=False),
        out_type=jax.ShapeDtypeStruct((NCHKS * NCHUNK, 640), jnp.float32),
        scratch_types=[
            pltpu.VMEM((64, 256), jnp.float32),
            pltpu.VMEM((64, 256), jnp.float32),
            pltpu.VMEM((64,), jnp.int32),
            pltpu.VMEM((64,), jnp.int32),
            pltpu.VMEM((64,), jnp.int32),
            pltpu.VMEM((NCHP,), jnp.int32),
            pltpu.VMEM((NCHUNK + 1, 640), jnp.float32),
            pltpu.SemaphoreType.DMA,
            pltpu.SemaphoreType.DMA,
            pltpu.SemaphoreType.DMA,
            pltpu.SemaphoreType.DMA,
        ],
    )
    def k3(pay_hbm, bins_hbm, cnt_hbm, zeros_hbm, acc_hbm,
           pay_a, pay_b, ids_a, ids_b, idx_v, cnt_v, acc_v,
           s_ia, s_ib, s_pa, s_pb):
        cid = lax.axis_index("c")
        sid = lax.axis_index("s")
        wid = sid * 2 + cid
        pltpu.sync_copy(cnt_hbm, cnt_v)
        lane = lax.broadcasted_iota(jnp.int32, (16,), 0)

        for ph in range(-(-NCHKS // 32)):
            chunk = ph * 32 + wid

            @pl.when(chunk < NCHKS)
            def _():
                abase = chunk * NCHUNK
                cnt = plsc.load_gather(cnt_v, [jnp.full((16,), chunk)])[0]
                pltpu.sync_copy(zeros_hbm, acc_v.at[pl.ds(0, NCHUNK)])

                def ids_slice(g):
                    return bins_hbm.at[pl.ds(chunk * CAPB + g * 64, 64)]

                def fire_ids(g, ids_v, sem):
                    @pl.when(g * 64 < cnt)
                    def _():
                        pltpu.async_copy(ids_slice(g), ids_v, sem)

                def recv_ids_fire_pay(g, ids_v, si, pay_v, sp):
                    @pl.when(g * 64 < cnt)
                    def _():
                        pltpu.make_async_copy(ids_slice(g), ids_v, si).wait()
                        for k in range(4):
                            sl = pl.ds(k * 16, 16)
                            gpos = lane + (g * 64 + k * 16)
                            okv = gpos < jnp.full((16,), cnt)
                            ids_v[sl] = jnp.where(okv, ids_v[sl], 0)
                        pltpu.async_copy(pay_hbm.at[ids_v], pay_v, sp)

                def recv_pay_expand(g, ids_v, pay_v, sp):
                    @pl.when(g * 64 < cnt)
                    def _():
                        pltpu.make_async_copy(
                            pay_hbm.at[ids_v], pay_v, sp).wait()
                        for k in range(4):
                            sl = pl.ds(k * 16, 16)
                            gpos = lane + (g * 64 + k * 16)
                            okv = gpos < jnp.full((16,), cnt)
                            i0g = plsc.bitcast(
                                plsc.load_gather(
                                    pay_v,
                                    [lane + k * 16, jnp.full((16,), 201)]),
                                jnp.int32)
                            idx_v[sl] = jnp.where(okv, i0g - abase, NCHUNK)

                        def edge(j, c2):
                            li = plsc.load_gather(idx_v, [jnp.full((16,), j)])
                            gvec = pay_v[j, pl.ds(192, 16)]
                            gv = [jnp.full((16,), gvec[t]) for t in range(9)]
                            cmid = [pay_v[j, pl.ds(64 + m * 16, 16)]
                                    for m in range(4)]
                            chi = [pay_v[j, pl.ds(128 + m * 16, 16)]
                                   for m in range(4)]
                            for m in range(4):
                                plsc.addupdate_scatter(
                                    acc_v,
                                    [li, lane + m * 16],
                                    pay_v[j, pl.ds(m * 16, 16)])
                            for p in range(1, 4):
                                for m in range(4):
                                    plsc.addupdate_scatter(
                                        acc_v,
                                        [li, lane + (p * 64 + m * 16)],
                                        cmid[m] * gv[p - 1])
                            for p in range(4, 10):
                                for m in range(4):
                                    plsc.addupdate_scatter(
                                        acc_v,
                                        [li, lane + (p * 64 + m * 16)],
                                        chi[m] * gv[p - 1])
                            return c2

                        lax.fori_loop(0, 64, edge, 0)

                fire_ids(0, ids_a, s_ia)
                recv_ids_fire_pay(0, ids_a, s_ia, pay_a, s_pa)
                fire_ids(1, ids_b, s_ib)

                def pair(p, carry):
                    g0 = p * 2
                    g1 = g0 + 1
                    recv_ids_fire_pay(g1, ids_b, s_ib, pay_b, s_pb)
                    recv_pay_expand(g0, ids_a, pay_a, s_pa)
                    fire_ids(g0 + 2, ids_a, s_ia)
                    recv_pay_expand(g1, ids_b, pay_b, s_pb)
                    fire_ids(g1 + 2, ids_b, s_ib)
                    recv_ids_fire_pay(g0 + 2, ids_a, s_ia, pay_a, s_pa)
                    return carry

                lax.fori_loop(0, MAXG // 2, pair, 0)
                pltpu.sync_copy(acc_v.at[pl.ds(0, NCHUNK)],
                                acc_hbm.at[pl.ds(abase, NCHUNK), :])

    return k3


def _k4_body(acc_ref, g_ref, b_ref, w1t_ref, b1_ref, w2t_ref, b2_ref,
             wit_ref, wat_ref, wst_ref, out_ref):
    A = acc_ref[...]
    x = [A[:, 64 * p:64 * (p + 1)] for p in range(10)]
    trace = x[4] + x[5] + x[6]
    dd = x[0] - trace * (1.0 / 3.0)
    norms = ((dd + x[4]) ** 2 + (dd + x[5]) ** 2 + (dd + x[6]) ** 2
             + 2.0 * (x[7] ** 2 + x[3] ** 2)
             + 2.0 * (x[8] ** 2 + x[2] ** 2)
             + 2.0 * (x[9] ** 2 + x[1] ** 2))
    mean = jnp.mean(norms, axis=-1, keepdims=True)
    var = jnp.mean((norms - mean) ** 2, axis=-1, keepdims=True)
    nn = (norms - mean) * lax.rsqrt(var + 1e-5) * g_ref[...] + b_ref[...]
    h1 = jnp.dot(nn, w1t_ref[...], preferred_element_type=jnp.float32)
    h1 = h1 + b1_ref[...]
    h1 = h1 * (1.0 / (1.0 + jnp.exp(-h1)))
    co = jnp.dot(h1, w2t_ref[...], preferred_element_type=jnp.float32)
    co = co + b2_ref[...]
    co = co * (1.0 / (1.0 + jnp.exp(-co)))
    cI2 = co[:, 0:64]
    cA2 = co[:, 64:128]
    cS2 = co[:, 128:192]
    ws = [wit_ref, wat_ref, wat_ref, wat_ref] + [wst_ref] * 6
    t = [jnp.dot(x[p], ws[p][...], preferred_element_type=jnp.float32)
         for p in range(10)]
    tr3 = (t[4] + t[5] + t[6]) * (1.0 / 3.0)
    diag = cI2 * t[0]
    out_ref[0] = diag + cS2 * (t[4] - tr3)
    out_ref[1] = cS2 * t[7] - cA2 * t[3]
    out_ref[2] = cS2 * t[8] + cA2 * t[2]
    out_ref[3] = cS2 * t[7] + cA2 * t[3]
    out_ref[4] = diag + cS2 * (t[5] - tr3)
    out_ref[5] = cS2 * t[9] - cA2 * t[1]
    out_ref[6] = cS2 * t[8] - cA2 * t[2]
    out_ref[7] = cS2 * t[9] + cA2 * t[1]
    out_ref[8] = diag + cS2 * (t[6] - tr3)


def _atom_stage(N, Acc, ln_g2, ln_b2, W1T, b12, W2T, b22, WIT, WAT, WST):
    BN = 1000
    return pl.pallas_call(
        _k4_body,
        grid=(N // BN,),
        in_specs=[
            pl.BlockSpec((BN, 640), lambda i: (i, 0)),
            pl.BlockSpec((1, EMB), lambda i: (0, 0)),
            pl.BlockSpec((1, EMB), lambda i: (0, 0)),
            pl.BlockSpec((EMB, 2 * EMB), lambda i: (0, 0)),
            pl.BlockSpec((1, 2 * EMB), lambda i: (0, 0)),
            pl.BlockSpec((2 * EMB, 3 * EMB), lambda i: (0, 0)),
            pl.BlockSpec((1, 3 * EMB), lambda i: (0, 0)),
            pl.BlockSpec((EMB, EMB), lambda i: (0, 0)),
            pl.BlockSpec((EMB, EMB), lambda i: (0, 0)),
            pl.BlockSpec((EMB, EMB), lambda i: (0, 0)),
        ],
        out_specs=pl.BlockSpec((9, BN, EMB), lambda i: (0, i, 0)),
        out_shape=jax.ShapeDtypeStruct((9, N, EMB), jnp.float32),
    )(Acc, ln_g2, ln_b2, W1T, b12, W2T, b22, WIT, WAT, WST)


def kernel(neighbour_vectors, Z, neighbour_index, z_table, Wz, Wr, br,
           ln_g, ln_b, W1, b1, W2, b2, WI, WA, WS):
    E = neighbour_vectors.shape[0]
    N = Z.shape[0]

    nvT = jnp.zeros((4, E), jnp.float32).at[:3].set(neighbour_vectors.T)
    i0 = neighbour_index[0]
    i1 = neighbour_index[1]
    idxA = jnp.concatenate([i0[0::2], i1[0::2]])
    idxB = jnp.concatenate([i0[1::2], i1[1::2]])

    P01 = _species_tables(Z[:, None], z_table,
                          Wz[:, :EMB].T, Wz[:, EMB:].T)
    Hsum = _make_k1(E)(P01, idxA, idxB)
    mu2 = jnp.linspace(jnp.exp(-CUTOFF), 1.0, RF,
                       dtype=jnp.float32)[:, None]
    i02 = i0[:, None]
    pay = _edge_payload(nvT, Hsum, Wr, br[:, None], mu2, i02)
    pos, cntf = _bin_positions(i02)
    bins = _make_bin_scatter(E)(pos[:, 0])
    zeros = jnp.zeros((NCHUNK, 640), jnp.float32)
    Acc = _make_k3(E, N)(pay, bins, cntf[0].astype(jnp.int32), zeros)
    X9 = _atom_stage(N, Acc, ln_g[None, :], ln_b[None, :], W1.T, b1[None, :],
                     W2.T, b2[None, :], WI.T, WA.T, WS.T)
    return jnp.transpose(X9, (1, 2, 0)).reshape(N, EMB, 3, 3)
